# baseline (speedup 1.0000x reference)
import functools
import jax
import jax.numpy as jnp
from jax import lax
from jax.experimental import pallas as pl
from jax.experimental.pallas import tpu as pltpu, tpu_sc as plsc

HIDDEN = 128
N_NODES = 10000
N_EDGES = 320000
SCALE = (12.0 / 6.0) / 100.0

NC = 2
NS = 16
NW = NC * NS
CHUNK = 128
NCHUNKS = N_EDGES // CHUNK
ITERS = (NCHUNKS + NW - 1) // NW
NP_PAD = 10240
ZROWS = NP_PAD // NS
TW = 8


def _nodeproj_body(h_ref, w1a_ref, w1b_ref, b1_ref, a_ref, b_ref):
    hb = h_ref[...]
    a = jnp.dot(hb, w1a_ref[...], preferred_element_type=jnp.float32) + b1_ref[...]
    b = jnp.dot(hb, w1b_ref[...], preferred_element_type=jnp.float32)
    a_ref[...] = a.astype(jnp.bfloat16)
    b_ref[...] = b.astype(jnp.bfloat16)


def _node_proj(h, w1a, w1b, b1r):
    blk = 2000
    grid = N_NODES // blk
    return pl.pallas_call(
        _nodeproj_body,
        grid=(grid,),
        in_specs=[
            pl.BlockSpec((blk, HIDDEN), lambda i: (i, 0)),
            pl.BlockSpec((HIDDEN, HIDDEN), lambda i: (0, 0)),
            pl.BlockSpec((HIDDEN, HIDDEN), lambda i: (0, 0)),
            pl.BlockSpec((1, HIDDEN), lambda i: (0, 0)),
        ],
        out_specs=[
            pl.BlockSpec((blk, HIDDEN), lambda i: (i, 0)),
            pl.BlockSpec((blk, HIDDEN), lambda i: (i, 0)),
        ],
        out_shape=[
            jax.ShapeDtypeStruct((N_NODES, HIDDEN), jnp.bfloat16),
            jax.ShapeDtypeStruct((N_NODES, HIDDEN), jnp.bfloat16),
        ],
    )(h, w1a, w1b, b1r)


GITERS = (NCHUNKS + NW - 1) // NW + ((NCHUNKS + NW - 1) // NW) % 2


def _gather_body(a_hbm, b_hbm, row_hbm, col_hbm, g1_hbm, g2_hbm,
                 i1a, i1b, i2a, i2b, r1a, r1b, r2a, r2b,
                 sia, sib, sga, sgb, swa, swb):
    w = lax.axis_index("s") * NC + lax.axis_index("c")

    idx_bufs = ((i1a, i2a), (i1b, i2b))
    row_bufs = ((r1a, r2a), (r1b, r2b))
    isems = (sia, sib)
    gsems = (sga, sgb)
    wsems = (swa, swb)

    def rd_base(j):
        c = w + NW * j
        return jnp.where(c < NCHUNKS, c, 0) * CHUNK

    def wr_base(j):
        c = w + NW * j
        return jnp.where(c < NCHUNKS, c * CHUNK, N_EDGES)

    def start_idx(j, b):
        base = rd_base(j)
        pltpu.async_copy(row_hbm.at[pl.ds(base, CHUNK)], idx_bufs[b][0], isems[b])
        pltpu.async_copy(col_hbm.at[pl.ds(base, CHUNK)], idx_bufs[b][1], isems[b])

    def wait_idx(b):
        pltpu.make_async_copy(row_hbm.at[pl.ds(0, CHUNK)], idx_bufs[b][0], isems[b]).wait()
        pltpu.make_async_copy(col_hbm.at[pl.ds(0, CHUNK)], idx_bufs[b][1], isems[b]).wait()

    def start_gather(b):
        pltpu.async_copy(a_hbm.at[idx_bufs[b][0]], row_bufs[b][0], gsems[b])
        pltpu.async_copy(b_hbm.at[idx_bufs[b][1]], row_bufs[b][1], gsems[b])

    def wait_gather(b):
        pltpu.make_async_copy(a_hbm.at[idx_bufs[b][0]], row_bufs[b][0], gsems[b]).wait()
        pltpu.make_async_copy(b_hbm.at[idx_bufs[b][1]], row_bufs[b][1], gsems[b]).wait()

    def start_write(j, b):
        base = wr_base(j)
        pltpu.async_copy(row_bufs[b][0], g1_hbm.at[pl.ds(base, CHUNK)], wsems[b])
        pltpu.async_copy(row_bufs[b][1], g2_hbm.at[pl.ds(base, CHUNK)], wsems[b])

    def wait_write(b):
        pltpu.make_async_copy(row_bufs[b][0], g1_hbm.at[pl.ds(0, CHUNK)], wsems[b]).wait()
        pltpu.make_async_copy(row_bufs[b][1], g2_hbm.at[pl.ds(0, CHUNK)], wsems[b]).wait()

    start_idx(0, 0)
    start_idx(1, 1)
    wait_idx(0)
    start_gather(0)
    start_write(GITERS, 1)

    def step(j2, carry):
        j = 2 * j2

        wait_idx(1)
        wait_write(1)
        start_gather(1)
        wait_gather(0)
        start_write(j, 0)
        start_idx(j + 2, 0)

        wait_idx(0)
        wait_write(0)
        start_gather(0)
        wait_gather(1)
        start_write(j + 1, 1)
        start_idx(j + 3, 1)

        return carry

    lax.fori_loop(0, GITERS // 2 - 1, step, 0)

    j = GITERS - 2
    wait_idx(1)
    wait_write(1)
    start_gather(1)
    wait_gather(0)
    start_write(j, 0)
    wait_gather(1)
    wait_write(0)
    start_write(j + 1, 1)
    wait_write(1)


def _sc_gather(a, b, row, col):
    mesh = plsc.VectorSubcoreMesh(core_axis_name="c", subcore_axis_name="s", num_cores=NC, num_subcores=NS)
    kern = pl.kernel(
        _gather_body,
        out_type=[
            jax.ShapeDtypeStruct((N_EDGES + CHUNK, HIDDEN), jnp.bfloat16),
            jax.ShapeDtypeStruct((N_EDGES + CHUNK, HIDDEN), jnp.bfloat16),
        ],
        mesh=mesh,
        compiler_params=pltpu.CompilerParams(use_tc_tiling_on_sc=False),
        scratch_types=[
            pltpu.VMEM((CHUNK,), jnp.int32),
            pltpu.VMEM((CHUNK,), jnp.int32),
            pltpu.VMEM((CHUNK,), jnp.int32),
            pltpu.VMEM((CHUNK,), jnp.int32),
            pltpu.VMEM((CHUNK, HIDDEN), jnp.bfloat16),
            pltpu.VMEM((CHUNK, HIDDEN), jnp.bfloat16),
            pltpu.VMEM((CHUNK, HIDDEN), jnp.bfloat16),
            pltpu.VMEM((CHUNK, HIDDEN), jnp.bfloat16),
            pltpu.SemaphoreType.DMA,
            pltpu.SemaphoreType.DMA,
            pltpu.SemaphoreType.DMA,
            pltpu.SemaphoreType.DMA,
            pltpu.SemaphoreType.DMA,
            pltpu.SemaphoreType.DMA,
        ],
    )
    return kern(a, b, row, col)


def _edgemlp_body(g1_ref, g2_ref, d_ref, do_ref, cd_ref,
                  w1d_ref, w1e_ref, w2_ref, b2_ref, w3_ref, out_ref):
    s = (g1_ref[...].astype(jnp.float32) + g2_ref[...].astype(jnp.float32)
         + d_ref[...] * w1d_ref[...]
         + do_ref[...] * w1e_ref[...])
    t1 = s * (1.0 / (1.0 + jnp.exp(-s)))
    t2p = jnp.dot(t1.astype(jnp.bfloat16), w2_ref[...],
                  preferred_element_type=jnp.float32) + b2_ref[...]
    t2 = t2p * (1.0 / (1.0 + jnp.exp(-t2p)))
    t3 = jnp.dot(t2.astype(jnp.bfloat16), w3_ref[...],
                 preferred_element_type=jnp.float32)
    out_ref[...] = cd_ref[...] * (jnp.tanh(t3) * SCALE)


def _edge_mlp(g1, g2, d, do_, cd8, w1d, w1e, W2, b2r, W3):
    blk = 1280
    grid = N_EDGES // blk
    return pl.pallas_call(
        _edgemlp_body,
        grid=(grid,),
        in_specs=[
            pl.BlockSpec((blk, HIDDEN), lambda i: (i, 0)),
            pl.BlockSpec((blk, HIDDEN), lambda i: (i, 0)),
            pl.BlockSpec((blk, 1), lambda i: (i, 0)),
            pl.BlockSpec((blk, 1), lambda i: (i, 0)),
            pl.BlockSpec((blk, TW), lambda i: (i, 0)),
            pl.BlockSpec((1, HIDDEN), lambda i: (0, 0)),
            pl.BlockSpec((1, HIDDEN), lambda i: (0, 0)),
            pl.BlockSpec((HIDDEN, HIDDEN), lambda i: (0, 0)),
            pl.BlockSpec((1, HIDDEN), lambda i: (0, 0)),
            pl.BlockSpec((HIDDEN, 1), lambda i: (0, 0)),
        ],
        out_specs=pl.BlockSpec((blk, TW), lambda i: (i, 0)),
        out_shape=jax.ShapeDtypeStruct((N_EDGES, TW), jnp.float32),
    )(g1, g2, d, do_, cd8, w1d, w1e, W2, b2r, W3)


def _scatter_body(trans_hbm, row_hbm, zero_hbm, out_hbm, idx_v, t_v, acc_v):
    cid = lax.axis_index("c")
    sid = lax.axis_index("s")
    w = sid * NC + cid

    pltpu.sync_copy(zero_hbm, acc_v)

    def step(j, carry):
        c = w + NW * j

        @pl.when(c < NCHUNKS)
        def _():
            base = c * CHUNK
            pltpu.sync_copy(row_hbm.at[pl.ds(base, CHUNK)], idx_v)
            pltpu.sync_copy(trans_hbm.at[pl.ds(base * TW, CHUNK * TW)], t_v)
            lane = lax.iota(jnp.int32, 16)
            for k in range(CHUNK // 16):
                e16 = (lane + (k * 16)) * TW
                row16 = idx_v[pl.ds(k * 16, 16)] * TW
                for comp in range(3):
                    vals = plsc.load_gather(t_v, [e16 + comp])
                    plsc.addupdate_scatter(acc_v, [row16 + comp], vals)

        return carry

    lax.fori_loop(0, ITERS, step, 0)

    pltpu.sync_copy(acc_v, out_hbm.at[pl.ds(w * NP_PAD * TW, NP_PAD * TW)])


def _sc_scatter(trans_flat, row, zeros_flat):
    mesh = plsc.VectorSubcoreMesh(core_axis_name="c", subcore_axis_name="s", num_cores=NC, num_subcores=NS)
    kern = pl.kernel(
        _scatter_body,
        out_type=jax.ShapeDtypeStruct((NW * NP_PAD * TW,), jnp.float32),
        mesh=mesh,
        compiler_params=pltpu.CompilerParams(needs_layout_passes=False),
        scratch_types=[
            pltpu.VMEM((CHUNK,), jnp.int32),
            pltpu.VMEM((CHUNK * TW,), jnp.float32),
            pltpu.VMEM((NP_PAD * TW,), jnp.float32),
        ],
    )
    return kern(trans_flat, row, zeros_flat)


def _reduce_body(p_ref, x8_ref, out_ref):
    out_ref[...] = x8_ref[...] + jnp.sum(p_ref[...], axis=0)


def _tc_reduce(partials, x8):
    blk = 1024
    grid = NP_PAD // blk
    return pl.pallas_call(
        _reduce_body,
        grid=(grid,),
        in_specs=[
            pl.BlockSpec((NW, blk, TW), lambda i: (0, i, 0)),
            pl.BlockSpec((blk, TW), lambda i: (i, 0)),
        ],
        out_specs=pl.BlockSpec((blk, TW), lambda i: (i, 0)),
        out_shape=jax.ShapeDtypeStruct((NP_PAD, TW), jnp.float32),
    )(partials, x8)


@jax.jit
def kernel(h, x, edges, coord_diff, distances, distance_org, W1, b1, W2, b2, W3):
    row = edges[0].astype(jnp.int32)
    col = edges[1].astype(jnp.int32)
    W2 = W2.astype(jnp.bfloat16)
    W3 = W3.astype(jnp.bfloat16)

    w1a = W1[:HIDDEN]
    w1b = W1[HIDDEN:2 * HIDDEN]
    w1d = W1[2 * HIDDEN].reshape(1, HIDDEN)
    w1e = W1[2 * HIDDEN + 1].reshape(1, HIDDEN)
    b1r = b1.reshape(1, HIDDEN)
    b2r = b2.reshape(1, HIDDEN)

    a, b = _node_proj(h, w1a, w1b, b1r)
    g1, g2 = _sc_gather(a, b, row, col)

    cd8 = jnp.pad(coord_diff, ((0, 0), (0, TW - 3)))
    trans = _edge_mlp(g1, g2, distances, distance_org, cd8, w1d, w1e, W2, b2r, W3)

    zeros_flat = jnp.zeros((NP_PAD * TW,), jnp.float32)
    partials = _sc_scatter(trans.reshape(-1), row, zeros_flat).reshape(NW, NP_PAD, TW)

    x8 = jnp.pad(x, ((0, NP_PAD - N_NODES), (0, TW - 3)))
    out = _tc_reduce(partials, x8)
    return out[:N_NODES, :3]


# speedup vs baseline: 1.4189x; 1.4189x over previous
import functools
import jax
import jax.numpy as jnp
from jax import lax
from jax.experimental import pallas as pl
from jax.experimental.pallas import tpu as pltpu, tpu_sc as plsc

HIDDEN = 128
N_NODES = 10000
N_EDGES = 320000
SCALE = (12.0 / 6.0) / 100.0

NC = 2
NS = 16
NW = NC * NS
CHUNK = 128
NCHUNKS = N_EDGES // CHUNK
ITERS = (NCHUNKS + NW - 1) // NW
NP_PAD = 10240
ZROWS = NP_PAD // NS
TW = 8


def _nodeproj_body(h_ref, w1a_ref, w1b_ref, b1_ref, a_ref, b_ref):
    hb = h_ref[...]
    a = jnp.dot(hb, w1a_ref[...], preferred_element_type=jnp.float32) + b1_ref[...]
    b = jnp.dot(hb, w1b_ref[...], preferred_element_type=jnp.float32)
    a_ref[...] = a
    b_ref[...] = b


def _node_proj(h, w1a, w1b, b1r):
    blk = 2000
    grid = N_NODES // blk
    return pl.pallas_call(
        _nodeproj_body,
        grid=(grid,),
        in_specs=[
            pl.BlockSpec((blk, HIDDEN), lambda i: (i, 0)),
            pl.BlockSpec((HIDDEN, HIDDEN), lambda i: (0, 0)),
            pl.BlockSpec((HIDDEN, HIDDEN), lambda i: (0, 0)),
            pl.BlockSpec((1, HIDDEN), lambda i: (0, 0)),
        ],
        out_specs=[
            pl.BlockSpec((blk, HIDDEN), lambda i: (i, 0)),
            pl.BlockSpec((blk, HIDDEN), lambda i: (i, 0)),
        ],
        out_shape=[
            jax.ShapeDtypeStruct((N_NODES, HIDDEN), jnp.float32),
            jax.ShapeDtypeStruct((N_NODES, HIDDEN), jnp.float32),
        ],
    )(h, w1a, w1b, b1r)


GITERS = (NCHUNKS + NW - 1) // NW + ((NCHUNKS + NW - 1) // NW) % 2


def _gather_body(a_hbm, b_hbm, row_hbm, col_hbm, g1_hbm, g2_hbm,
                 i1a, i1b, i2a, i2b, r1a, r1b, r2a, r2b,
                 sia, sib, sga, sgb, swa, swb):
    w = lax.axis_index("s") * NC + lax.axis_index("c")

    idx_bufs = ((i1a, i2a), (i1b, i2b))
    row_bufs = ((r1a, r2a), (r1b, r2b))
    isems = (sia, sib)
    gsems = (sga, sgb)
    wsems = (swa, swb)

    def rd_base(j):
        c = w + NW * j
        return jnp.where(c < NCHUNKS, c, 0) * CHUNK

    def wr_base(j):
        c = w + NW * j
        return jnp.where(c < NCHUNKS, c * CHUNK, N_EDGES)

    def start_idx(j, b):
        base = rd_base(j)
        pltpu.async_copy(row_hbm.at[pl.ds(base, CHUNK)], idx_bufs[b][0], isems[b])
        pltpu.async_copy(col_hbm.at[pl.ds(base, CHUNK)], idx_bufs[b][1], isems[b])

    def wait_idx(b):
        pltpu.make_async_copy(row_hbm.at[pl.ds(0, CHUNK)], idx_bufs[b][0], isems[b]).wait()
        pltpu.make_async_copy(col_hbm.at[pl.ds(0, CHUNK)], idx_bufs[b][1], isems[b]).wait()

    def start_gather(b):
        pltpu.async_copy(a_hbm.at[idx_bufs[b][0]], row_bufs[b][0], gsems[b])
        pltpu.async_copy(b_hbm.at[idx_bufs[b][1]], row_bufs[b][1], gsems[b])

    def wait_gather(b):
        pltpu.make_async_copy(a_hbm.at[idx_bufs[b][0]], row_bufs[b][0], gsems[b]).wait()
        pltpu.make_async_copy(b_hbm.at[idx_bufs[b][1]], row_bufs[b][1], gsems[b]).wait()

    def start_write(j, b):
        base = wr_base(j)
        pltpu.async_copy(row_bufs[b][0], g1_hbm.at[pl.ds(base, CHUNK)], wsems[b])
        pltpu.async_copy(row_bufs[b][1], g2_hbm.at[pl.ds(base, CHUNK)], wsems[b])

    def wait_write(b):
        pltpu.make_async_copy(row_bufs[b][0], g1_hbm.at[pl.ds(0, CHUNK)], wsems[b]).wait()
        pltpu.make_async_copy(row_bufs[b][1], g2_hbm.at[pl.ds(0, CHUNK)], wsems[b]).wait()

    start_idx(0, 0)
    start_idx(1, 1)
    wait_idx(0)
    start_gather(0)
    start_write(GITERS, 1)

    def step(j2, carry):
        j = 2 * j2

        wait_idx(1)
        wait_write(1)
        start_gather(1)
        wait_gather(0)
        start_write(j, 0)
        start_idx(j + 2, 0)

        wait_idx(0)
        wait_write(0)
        start_gather(0)
        wait_gather(1)
        start_write(j + 1, 1)
        start_idx(j + 3, 1)

        return carry

    lax.fori_loop(0, GITERS // 2 - 1, step, 0)

    j = GITERS - 2
    wait_idx(1)
    wait_write(1)
    start_gather(1)
    wait_gather(0)
    start_write(j, 0)
    wait_gather(1)
    wait_write(0)
    start_write(j + 1, 1)
    wait_write(1)


def _sc_gather(a, b, row, col):
    mesh = plsc.VectorSubcoreMesh(core_axis_name="c", subcore_axis_name="s", num_cores=NC, num_subcores=NS)
    kern = pl.kernel(
        _gather_body,
        out_type=[
            jax.ShapeDtypeStruct((N_EDGES + CHUNK, HIDDEN), jnp.float32),
            jax.ShapeDtypeStruct((N_EDGES + CHUNK, HIDDEN), jnp.float32),
        ],
        mesh=mesh,
        scratch_types=[
            pltpu.VMEM((CHUNK,), jnp.int32),
            pltpu.VMEM((CHUNK,), jnp.int32),
            pltpu.VMEM((CHUNK,), jnp.int32),
            pltpu.VMEM((CHUNK,), jnp.int32),
            pltpu.VMEM((CHUNK, HIDDEN), jnp.float32),
            pltpu.VMEM((CHUNK, HIDDEN), jnp.float32),
            pltpu.VMEM((CHUNK, HIDDEN), jnp.float32),
            pltpu.VMEM((CHUNK, HIDDEN), jnp.float32),
            pltpu.SemaphoreType.DMA,
            pltpu.SemaphoreType.DMA,
            pltpu.SemaphoreType.DMA,
            pltpu.SemaphoreType.DMA,
            pltpu.SemaphoreType.DMA,
            pltpu.SemaphoreType.DMA,
        ],
    )
    return kern(a, b, row, col)


def _edgemlp_body(g1_ref, g2_ref, d_ref, do_ref, cd_ref,
                  w1d_ref, w1e_ref, w2_ref, b2_ref, w3_ref, out_ref):
    s = (g1_ref[...] + g2_ref[...]
         + d_ref[...] * w1d_ref[...]
         + do_ref[...] * w1e_ref[...])
    t1 = s * (1.0 / (1.0 + jnp.exp(-s)))
    t2p = jnp.dot(t1.astype(jnp.bfloat16), w2_ref[...],
                  preferred_element_type=jnp.float32) + b2_ref[...]
    t2 = t2p * (1.0 / (1.0 + jnp.exp(-t2p)))
    t3 = jnp.dot(t2.astype(jnp.bfloat16), w3_ref[...],
                 preferred_element_type=jnp.float32)
    out_ref[...] = cd_ref[...] * (jnp.tanh(t3) * SCALE)


def _edge_mlp(g1, g2, d, do_, cd8, w1d, w1e, W2, b2r, W3):
    blk = 1280
    grid = N_EDGES // blk
    return pl.pallas_call(
        _edgemlp_body,
        grid=(grid,),
        in_specs=[
            pl.BlockSpec((blk, HIDDEN), lambda i: (i, 0)),
            pl.BlockSpec((blk, HIDDEN), lambda i: (i, 0)),
            pl.BlockSpec((blk, 1), lambda i: (i, 0)),
            pl.BlockSpec((blk, 1), lambda i: (i, 0)),
            pl.BlockSpec((blk, TW), lambda i: (i, 0)),
            pl.BlockSpec((1, HIDDEN), lambda i: (0, 0)),
            pl.BlockSpec((1, HIDDEN), lambda i: (0, 0)),
            pl.BlockSpec((HIDDEN, HIDDEN), lambda i: (0, 0)),
            pl.BlockSpec((1, HIDDEN), lambda i: (0, 0)),
            pl.BlockSpec((HIDDEN, 1), lambda i: (0, 0)),
        ],
        out_specs=pl.BlockSpec((blk, TW), lambda i: (i, 0)),
        out_shape=jax.ShapeDtypeStruct((N_EDGES, TW), jnp.float32),
    )(g1, g2, d, do_, cd8, w1d, w1e, W2, b2r, W3)


def _scatter_body(trans_hbm, row_hbm, zero_hbm, out_hbm, idx_v, t_v, acc_v):
    cid = lax.axis_index("c")
    sid = lax.axis_index("s")
    w = sid * NC + cid

    pltpu.sync_copy(zero_hbm, acc_v)

    def step(j, carry):
        c = w + NW * j

        @pl.when(c < NCHUNKS)
        def _():
            base = c * CHUNK
            pltpu.sync_copy(row_hbm.at[pl.ds(base, CHUNK)], idx_v)
            pltpu.sync_copy(trans_hbm.at[pl.ds(base * TW, CHUNK * TW)], t_v)
            lane = lax.iota(jnp.int32, 16)
            for k in range(CHUNK // 16):
                e16 = (lane + (k * 16)) * TW
                row16 = idx_v[pl.ds(k * 16, 16)] * TW
                for comp in range(3):
                    vals = plsc.load_gather(t_v, [e16 + comp])
                    plsc.addupdate_scatter(acc_v, [row16 + comp], vals)

        return carry

    lax.fori_loop(0, ITERS, step, 0)

    pltpu.sync_copy(acc_v, out_hbm.at[pl.ds(w * NP_PAD * TW, NP_PAD * TW)])


def _sc_scatter(trans_flat, row, zeros_flat):
    mesh = plsc.VectorSubcoreMesh(core_axis_name="c", subcore_axis_name="s", num_cores=NC, num_subcores=NS)
    kern = pl.kernel(
        _scatter_body,
        out_type=jax.ShapeDtypeStruct((NW * NP_PAD * TW,), jnp.float32),
        mesh=mesh,
        compiler_params=pltpu.CompilerParams(needs_layout_passes=False),
        scratch_types=[
            pltpu.VMEM((CHUNK,), jnp.int32),
            pltpu.VMEM((CHUNK * TW,), jnp.float32),
            pltpu.VMEM((NP_PAD * TW,), jnp.float32),
        ],
    )
    return kern(trans_flat, row, zeros_flat)


def _reduce_body(p_ref, x8_ref, out_ref):
    out_ref[...] = x8_ref[...] + jnp.sum(p_ref[...], axis=0)


def _tc_reduce(partials, x8):
    blk = 1024
    grid = NP_PAD // blk
    return pl.pallas_call(
        _reduce_body,
        grid=(grid,),
        in_specs=[
            pl.BlockSpec((NW, blk, TW), lambda i: (0, i, 0)),
            pl.BlockSpec((blk, TW), lambda i: (i, 0)),
        ],
        out_specs=pl.BlockSpec((blk, TW), lambda i: (i, 0)),
        out_shape=jax.ShapeDtypeStruct((NP_PAD, TW), jnp.float32),
    )(partials, x8)


@jax.jit
def kernel(h, x, edges, coord_diff, distances, distance_org, W1, b1, W2, b2, W3):
    row = edges[0].astype(jnp.int32)
    col = edges[1].astype(jnp.int32)
    W2 = W2.astype(jnp.bfloat16)
    W3 = W3.astype(jnp.bfloat16)

    w1a = W1[:HIDDEN]
    w1b = W1[HIDDEN:2 * HIDDEN]
    w1d = W1[2 * HIDDEN].reshape(1, HIDDEN)
    w1e = W1[2 * HIDDEN + 1].reshape(1, HIDDEN)
    b1r = b1.reshape(1, HIDDEN)
    b2r = b2.reshape(1, HIDDEN)

    a, b = _node_proj(h, w1a, w1b, b1r)
    g1, g2 = _sc_gather(a, b, row, col)

    cd8 = jnp.pad(coord_diff, ((0, 0), (0, TW - 3)))
    trans = _edge_mlp(g1, g2, distances, distance_org, cd8, w1d, w1e, W2, b2r, W3)

    zeros_flat = jnp.zeros((NP_PAD * TW,), jnp.float32)
    partials = _sc_scatter(trans.reshape(-1), row, zeros_flat).reshape(NW, NP_PAD, TW)

    x8 = jnp.pad(x, ((0, NP_PAD - N_NODES), (0, TW - 3)))
    out = _tc_reduce(partials, x8)
    return out[:N_NODES, :3]


# speedup vs baseline: 2.6502x; 1.8678x over previous
import functools
import jax
import jax.numpy as jnp
from jax import lax
from jax.experimental import pallas as pl
from jax.experimental.pallas import tpu as pltpu, tpu_sc as plsc

HIDDEN = 128
N_NODES = 10000
N_EDGES = 320000
SCALE = (12.0 / 6.0) / 100.0

NC = 2
NS = 16
NW = NC * NS
CHUNK = 128
NCHUNKS = N_EDGES // CHUNK
ITERS = (NCHUNKS + NW - 1) // NW
NP_PAD = 10240
ZROWS = NP_PAD // NS
TW = 8


def _nodeproj_body(h_ref, w1a_ref, w1b_ref, b1_ref, a_ref, b_ref):
    hb = h_ref[...]
    a = jnp.dot(hb, w1a_ref[...], preferred_element_type=jnp.float32) + b1_ref[...]
    b = jnp.dot(hb, w1b_ref[...], preferred_element_type=jnp.float32)
    a_ref[...] = a
    b_ref[...] = b


def _node_proj(h, w1a, w1b, b1r):
    blk = 2000
    grid = N_NODES // blk
    return pl.pallas_call(
        _nodeproj_body,
        grid=(grid,),
        in_specs=[
            pl.BlockSpec((blk, HIDDEN), lambda i: (i, 0)),
            pl.BlockSpec((HIDDEN, HIDDEN), lambda i: (0, 0)),
            pl.BlockSpec((HIDDEN, HIDDEN), lambda i: (0, 0)),
            pl.BlockSpec((1, HIDDEN), lambda i: (0, 0)),
        ],
        out_specs=[
            pl.BlockSpec((blk, HIDDEN), lambda i: (i, 0)),
            pl.BlockSpec((blk, HIDDEN), lambda i: (i, 0)),
        ],
        out_shape=[
            jax.ShapeDtypeStruct((N_NODES, HIDDEN), jnp.float32),
            jax.ShapeDtypeStruct((N_NODES, HIDDEN), jnp.float32),
        ],
    )(h, w1a, w1b, b1r)


GITERS = (NCHUNKS + NW - 1) // NW + ((NCHUNKS + NW - 1) // NW) % 2


def _gather_body(a_hbm, b_hbm, row_hbm, col_hbm, g1_hbm, g2_hbm,
                 i1a, i1b, i2a, i2b, r1a, r1b, r2a, r2b,
                 sia, sib, sga, sgb, swa, swb):
    w = lax.axis_index("s") * NC + lax.axis_index("c")

    idx_bufs = ((i1a, i2a), (i1b, i2b))
    row_bufs = ((r1a, r2a), (r1b, r2b))
    isems = (sia, sib)
    gsems = (sga, sgb)
    wsems = (swa, swb)

    def rd_base(j):
        c = w + NW * j
        return jnp.where(c < NCHUNKS, c, 0) * CHUNK

    def wr_base(j):
        c = w + NW * j
        return jnp.where(c < NCHUNKS, c * CHUNK, N_EDGES)

    def start_idx(j, b):
        base = rd_base(j)
        pltpu.async_copy(row_hbm.at[pl.ds(base, CHUNK)], idx_bufs[b][0], isems[b])
        pltpu.async_copy(col_hbm.at[pl.ds(base, CHUNK)], idx_bufs[b][1], isems[b])

    def wait_idx(b):
        pltpu.make_async_copy(row_hbm.at[pl.ds(0, CHUNK)], idx_bufs[b][0], isems[b]).wait()
        pltpu.make_async_copy(col_hbm.at[pl.ds(0, CHUNK)], idx_bufs[b][1], isems[b]).wait()

    def start_gather(b):
        pltpu.async_copy(a_hbm.at[idx_bufs[b][0]], row_bufs[b][0], gsems[b])
        pltpu.async_copy(b_hbm.at[idx_bufs[b][1]], row_bufs[b][1], gsems[b])

    def wait_gather(b):
        pltpu.make_async_copy(a_hbm.at[idx_bufs[b][0]], row_bufs[b][0], gsems[b]).wait()
        pltpu.make_async_copy(b_hbm.at[idx_bufs[b][1]], row_bufs[b][1], gsems[b]).wait()

    def start_write(j, b):
        base = wr_base(j)
        pltpu.async_copy(row_bufs[b][0], g1_hbm.at[pl.ds(base, CHUNK)], wsems[b])
        pltpu.async_copy(row_bufs[b][1], g2_hbm.at[pl.ds(base, CHUNK)], wsems[b])

    def wait_write(b):
        pltpu.make_async_copy(row_bufs[b][0], g1_hbm.at[pl.ds(0, CHUNK)], wsems[b]).wait()
        pltpu.make_async_copy(row_bufs[b][1], g2_hbm.at[pl.ds(0, CHUNK)], wsems[b]).wait()

    start_idx(0, 0)
    start_idx(1, 1)
    wait_idx(0)
    start_gather(0)
    start_write(GITERS, 1)

    def step(j2, carry):
        j = 2 * j2

        wait_idx(1)
        wait_write(1)
        start_gather(1)
        wait_gather(0)
        start_write(j, 0)
        start_idx(j + 2, 0)

        wait_idx(0)
        wait_write(0)
        start_gather(0)
        wait_gather(1)
        start_write(j + 1, 1)
        start_idx(j + 3, 1)

        return carry

    lax.fori_loop(0, GITERS // 2 - 1, step, 0)

    j = GITERS - 2
    wait_idx(1)
    wait_write(1)
    start_gather(1)
    wait_gather(0)
    start_write(j, 0)
    wait_gather(1)
    wait_write(0)
    start_write(j + 1, 1)
    wait_write(1)


def _sc_gather(a, b, row, col):
    mesh = plsc.VectorSubcoreMesh(core_axis_name="c", subcore_axis_name="s", num_cores=NC, num_subcores=NS)
    kern = pl.kernel(
        _gather_body,
        out_type=[
            jax.ShapeDtypeStruct((N_EDGES + CHUNK, HIDDEN), jnp.float32),
            jax.ShapeDtypeStruct((N_EDGES + CHUNK, HIDDEN), jnp.float32),
        ],
        mesh=mesh,
        scratch_types=[
            pltpu.VMEM((CHUNK,), jnp.int32),
            pltpu.VMEM((CHUNK,), jnp.int32),
            pltpu.VMEM((CHUNK,), jnp.int32),
            pltpu.VMEM((CHUNK,), jnp.int32),
            pltpu.VMEM((CHUNK, HIDDEN), jnp.float32),
            pltpu.VMEM((CHUNK, HIDDEN), jnp.float32),
            pltpu.VMEM((CHUNK, HIDDEN), jnp.float32),
            pltpu.VMEM((CHUNK, HIDDEN), jnp.float32),
            pltpu.SemaphoreType.DMA,
            pltpu.SemaphoreType.DMA,
            pltpu.SemaphoreType.DMA,
            pltpu.SemaphoreType.DMA,
            pltpu.SemaphoreType.DMA,
            pltpu.SemaphoreType.DMA,
        ],
    )
    return kern(a, b, row, col)


EBLK = 1280
DROWS = EBLK // HIDDEN


def _edgemlp_body(g1_ref, g2_ref, d_ref, do_ref, cdt_ref,
                  w1d_ref, w1e_ref, w2_ref, b2_ref, w3_ref, eye_ref, out_ref):
    eye = eye_ref[...]
    dt = lax.dot_general(eye, d_ref[0], (((1,), (1,)), ((), ())),
                         preferred_element_type=jnp.float32)
    dot_ = lax.dot_general(eye, do_ref[0], (((1,), (1,)), ((), ())),
                           preferred_element_type=jnp.float32)
    dcol = jnp.concatenate([dt[:, r:r + 1] for r in range(DROWS)], axis=0)
    docol = jnp.concatenate([dot_[:, r:r + 1] for r in range(DROWS)], axis=0)
    s = (g1_ref[...] + g2_ref[...]
         + dcol * w1d_ref[...]
         + docol * w1e_ref[...])
    t1 = s * (1.0 / (1.0 + jnp.exp(-s)))
    t2p = jnp.dot(t1.astype(jnp.bfloat16), w2_ref[...],
                  preferred_element_type=jnp.float32) + b2_ref[...]
    t2 = t2p * (1.0 / (1.0 + jnp.exp(-t2p)))
    t3t = lax.dot_general(w3_ref[...], t2.astype(jnp.bfloat16),
                          (((0,), (1,)), ((), ())),
                          preferred_element_type=jnp.float32)
    out_ref[...] = cdt_ref[...] * (jnp.tanh(t3t) * SCALE)


def _edge_mlp(g1, g2, d2d, do2d, cdt, w1d, w1e, W2, b2r, W3):
    grid = N_EDGES // EBLK
    return pl.pallas_call(
        _edgemlp_body,
        grid=(grid,),
        in_specs=[
            pl.BlockSpec((EBLK, HIDDEN), lambda i: (i, 0)),
            pl.BlockSpec((EBLK, HIDDEN), lambda i: (i, 0)),
            pl.BlockSpec((1, DROWS, HIDDEN), lambda i: (i, 0, 0)),
            pl.BlockSpec((1, DROWS, HIDDEN), lambda i: (i, 0, 0)),
            pl.BlockSpec((3, EBLK), lambda i: (0, i)),
            pl.BlockSpec((1, HIDDEN), lambda i: (0, 0)),
            pl.BlockSpec((1, HIDDEN), lambda i: (0, 0)),
            pl.BlockSpec((HIDDEN, HIDDEN), lambda i: (0, 0)),
            pl.BlockSpec((1, HIDDEN), lambda i: (0, 0)),
            pl.BlockSpec((HIDDEN, 1), lambda i: (0, 0)),
            pl.BlockSpec((HIDDEN, HIDDEN), lambda i: (0, 0)),
        ],
        out_specs=pl.BlockSpec((3, EBLK), lambda i: (0, i)),
        out_shape=jax.ShapeDtypeStruct((3, N_EDGES), jnp.float32),
    )(g1, g2, d2d, do2d, cdt, w1d, w1e, W2, b2r, W3,
      jnp.eye(HIDDEN, dtype=jnp.float32))


def _scatter_body(trans_hbm, row_hbm, zero_hbm, out_hbm, idx_v, t_v, acc_v):
    cid = lax.axis_index("c")
    sid = lax.axis_index("s")
    w = sid * NC + cid

    pltpu.sync_copy(zero_hbm, acc_v)

    def step(j, carry):
        c = w + NW * j

        @pl.when(c < NCHUNKS)
        def _():
            base = c * CHUNK
            pltpu.sync_copy(row_hbm.at[pl.ds(base, CHUNK)], idx_v)
            pltpu.sync_copy(trans_hbm.at[:, pl.ds(base, CHUNK)], t_v)
            lane = lax.iota(jnp.int32, 16)
            for k in range(CHUNK // 16):
                row16 = idx_v[pl.ds(k * 16, 16)]
                for comp in range(3):
                    vals = t_v[comp, pl.ds(k * 16, 16)]
                    plsc.addupdate_scatter(
                        acc_v, [row16 + (comp * NP_PAD)], vals)

        return carry

    lax.fori_loop(0, ITERS, step, 0)

    pltpu.sync_copy(acc_v, out_hbm.at[pl.ds(w * 3 * NP_PAD, 3 * NP_PAD)])


def _sc_scatter(trans, row, zeros_flat):
    mesh = plsc.VectorSubcoreMesh(core_axis_name="c", subcore_axis_name="s", num_cores=NC, num_subcores=NS)
    kern = pl.kernel(
        _scatter_body,
        out_type=jax.ShapeDtypeStruct((NW * 3 * NP_PAD,), jnp.float32),
        mesh=mesh,
        compiler_params=pltpu.CompilerParams(needs_layout_passes=False),
        scratch_types=[
            pltpu.VMEM((CHUNK,), jnp.int32),
            pltpu.VMEM((3, CHUNK), jnp.float32),
            pltpu.VMEM((3 * NP_PAD,), jnp.float32),
        ],
    )
    return kern(trans, row, zeros_flat)


def _reduce_body(p_ref, xt_ref, out_ref):
    out_ref[...] = xt_ref[...] + jnp.sum(p_ref[...], axis=0)


def _tc_reduce(partials, xt):
    blk = 2048
    grid = NP_PAD // blk
    return pl.pallas_call(
        _reduce_body,
        grid=(grid,),
        in_specs=[
            pl.BlockSpec((NW, 3, blk), lambda i: (0, 0, i)),
            pl.BlockSpec((3, blk), lambda i: (0, i)),
        ],
        out_specs=pl.BlockSpec((3, blk), lambda i: (0, i)),
        out_shape=jax.ShapeDtypeStruct((3, NP_PAD), jnp.float32),
    )(partials, xt)


@jax.jit
def kernel(h, x, edges, coord_diff, distances, distance_org, W1, b1, W2, b2, W3):
    row = edges[0].astype(jnp.int32)
    col = edges[1].astype(jnp.int32)
    W2 = W2.astype(jnp.bfloat16)
    W3 = W3.astype(jnp.bfloat16)

    w1a = W1[:HIDDEN]
    w1b = W1[HIDDEN:2 * HIDDEN]
    w1d = W1[2 * HIDDEN].reshape(1, HIDDEN)
    w1e = W1[2 * HIDDEN + 1].reshape(1, HIDDEN)
    b1r = b1.reshape(1, HIDDEN)
    b2r = b2.reshape(1, HIDDEN)

    a, b = _node_proj(h, w1a, w1b, b1r)
    g1, g2 = _sc_gather(a, b, row, col)

    d2d = distances.reshape(N_EDGES // EBLK, DROWS, HIDDEN)
    do2d = distance_org.reshape(N_EDGES // EBLK, DROWS, HIDDEN)
    cdt = coord_diff.T
    trans = _edge_mlp(g1, g2, d2d, do2d, cdt, w1d, w1e, W2, b2r, W3)

    zeros_flat = jnp.zeros((3 * NP_PAD,), jnp.float32)
    partials = _sc_scatter(trans, row, zeros_flat)

    xt = jnp.pad(x.T, ((0, 0), (0, NP_PAD - N_NODES)))
    out = _tc_reduce(partials.reshape(NW, 3, NP_PAD), xt)
    return out[:, :N_NODES].T


# speedup vs baseline: 2.9289x; 1.1052x over previous
import functools
import jax
import jax.numpy as jnp
from jax import lax
from jax.experimental import pallas as pl
from jax.experimental.pallas import tpu as pltpu, tpu_sc as plsc

HIDDEN = 128
N_NODES = 10000
N_EDGES = 320000
SCALE = (12.0 / 6.0) / 100.0

NC = 2
NS = 16
NW = NC * NS
CHUNK = 128
NCHUNKS = N_EDGES // CHUNK
ITERS = (NCHUNKS + NW - 1) // NW
NP_PAD = 10240
ZROWS = NP_PAD // NS
TW = 8


def _nodeproj_body(h_ref, w1a_ref, w1b_ref, b1_ref, a_ref, b_ref):
    hb = h_ref[...]
    a = jnp.dot(hb, w1a_ref[...], preferred_element_type=jnp.float32) + b1_ref[...]
    b = jnp.dot(hb, w1b_ref[...], preferred_element_type=jnp.float32)
    a_ref[...] = a
    b_ref[...] = b


def _node_proj(h, w1a, w1b, b1r):
    blk = 2000
    grid = N_NODES // blk
    return pl.pallas_call(
        _nodeproj_body,
        grid=(grid,),
        in_specs=[
            pl.BlockSpec((blk, HIDDEN), lambda i: (i, 0)),
            pl.BlockSpec((HIDDEN, HIDDEN), lambda i: (0, 0)),
            pl.BlockSpec((HIDDEN, HIDDEN), lambda i: (0, 0)),
            pl.BlockSpec((1, HIDDEN), lambda i: (0, 0)),
        ],
        out_specs=[
            pl.BlockSpec((blk, HIDDEN), lambda i: (i, 0)),
            pl.BlockSpec((blk, HIDDEN), lambda i: (i, 0)),
        ],
        out_shape=[
            jax.ShapeDtypeStruct((N_NODES, HIDDEN), jnp.float32),
            jax.ShapeDtypeStruct((N_NODES, HIDDEN), jnp.float32),
        ],
    )(h, w1a, w1b, b1r)


SEG = 5
E_SEG = N_EDGES // SEG
NCHUNKS_S = E_SEG // CHUNK
_G = (NCHUNKS_S + NW - 1) // NW
GITERS_S = _G + (_G % 2)


def _gather_body(a_hbm, b_hbm, row_hbm, col_hbm, g1_hbm, g2_hbm,
                 i1a, i1b, i2a, i2b, r1a, r1b, r2a, r2b,
                 sia, sib, sga, sgb, swa, swb):
    w = lax.axis_index("s") * NC + lax.axis_index("c")

    idx_bufs = ((i1a, i2a), (i1b, i2b))
    row_bufs = ((r1a, r2a), (r1b, r2b))
    isems = (sia, sib)
    gsems = (sga, sgb)
    wsems = (swa, swb)

    def rd_base(j):
        c = w + NW * j
        return jnp.where(c < NCHUNKS_S, c, 0) * CHUNK

    def wr_base(j):
        c = w + NW * j
        return jnp.where(c < NCHUNKS_S, c * CHUNK, E_SEG)

    def start_idx(j, b):
        base = rd_base(j)
        pltpu.async_copy(row_hbm.at[pl.ds(base, CHUNK)], idx_bufs[b][0], isems[b])
        pltpu.async_copy(col_hbm.at[pl.ds(base, CHUNK)], idx_bufs[b][1], isems[b])

    def wait_idx(b):
        pltpu.make_async_copy(row_hbm.at[pl.ds(0, CHUNK)], idx_bufs[b][0], isems[b]).wait()
        pltpu.make_async_copy(col_hbm.at[pl.ds(0, CHUNK)], idx_bufs[b][1], isems[b]).wait()

    def start_gather(b):
        pltpu.async_copy(a_hbm.at[idx_bufs[b][0]], row_bufs[b][0], gsems[b])
        pltpu.async_copy(b_hbm.at[idx_bufs[b][1]], row_bufs[b][1], gsems[b])

    def wait_gather(b):
        pltpu.make_async_copy(a_hbm.at[idx_bufs[b][0]], row_bufs[b][0], gsems[b]).wait()
        pltpu.make_async_copy(b_hbm.at[idx_bufs[b][1]], row_bufs[b][1], gsems[b]).wait()

    def start_write(j, b):
        base = wr_base(j)
        pltpu.async_copy(row_bufs[b][0], g1_hbm.at[pl.ds(base, CHUNK)], wsems[b])
        pltpu.async_copy(row_bufs[b][1], g2_hbm.at[pl.ds(base, CHUNK)], wsems[b])

    def wait_write(b):
        pltpu.make_async_copy(row_bufs[b][0], g1_hbm.at[pl.ds(0, CHUNK)], wsems[b]).wait()
        pltpu.make_async_copy(row_bufs[b][1], g2_hbm.at[pl.ds(0, CHUNK)], wsems[b]).wait()

    start_idx(0, 0)
    start_idx(1, 1)
    wait_idx(0)
    start_gather(0)
    start_write(GITERS_S, 1)

    def step(j2, carry):
        j = 2 * j2

        wait_idx(1)
        wait_write(1)
        start_gather(1)
        wait_gather(0)
        start_write(j, 0)
        start_idx(j + 2, 0)

        wait_idx(0)
        wait_write(0)
        start_gather(0)
        wait_gather(1)
        start_write(j + 1, 1)
        start_idx(j + 3, 1)

        return carry

    lax.fori_loop(0, GITERS_S // 2 - 1, step, 0)

    j = GITERS_S - 2
    wait_idx(1)
    wait_write(1)
    start_gather(1)
    wait_gather(0)
    start_write(j, 0)
    wait_gather(1)
    wait_write(0)
    start_write(j + 1, 1)
    wait_write(1)


def _sc_gather(a, b, row, col):
    mesh = plsc.VectorSubcoreMesh(core_axis_name="c", subcore_axis_name="s", num_cores=NC, num_subcores=NS)
    kern = pl.kernel(
        _gather_body,
        out_type=[
            jax.ShapeDtypeStruct((E_SEG + CHUNK, HIDDEN), jnp.float32),
            jax.ShapeDtypeStruct((E_SEG + CHUNK, HIDDEN), jnp.float32),
        ],
        mesh=mesh,
        scratch_types=[
            pltpu.VMEM((CHUNK,), jnp.int32),
            pltpu.VMEM((CHUNK,), jnp.int32),
            pltpu.VMEM((CHUNK,), jnp.int32),
            pltpu.VMEM((CHUNK,), jnp.int32),
            pltpu.VMEM((CHUNK, HIDDEN), jnp.float32),
            pltpu.VMEM((CHUNK, HIDDEN), jnp.float32),
            pltpu.VMEM((CHUNK, HIDDEN), jnp.float32),
            pltpu.VMEM((CHUNK, HIDDEN), jnp.float32),
            pltpu.SemaphoreType.DMA,
            pltpu.SemaphoreType.DMA,
            pltpu.SemaphoreType.DMA,
            pltpu.SemaphoreType.DMA,
            pltpu.SemaphoreType.DMA,
            pltpu.SemaphoreType.DMA,
        ],
    )
    return kern(a, b, row, col)


EBLK = 1280
DROWS = EBLK // HIDDEN


def _edgemlp_body(g1_ref, g2_ref, d_ref, do_ref, cdt_ref,
                  w1d_ref, w1e_ref, w2_ref, b2_ref, w3_ref, eye_ref, out_ref):
    eye = eye_ref[...]
    dt = lax.dot_general(eye, d_ref[0], (((1,), (1,)), ((), ())),
                         preferred_element_type=jnp.float32)
    dot_ = lax.dot_general(eye, do_ref[0], (((1,), (1,)), ((), ())),
                           preferred_element_type=jnp.float32)
    dcol = jnp.concatenate([dt[:, r:r + 1] for r in range(DROWS)], axis=0)
    docol = jnp.concatenate([dot_[:, r:r + 1] for r in range(DROWS)], axis=0)
    s = (g1_ref[...] + g2_ref[...]
         + dcol * w1d_ref[...]
         + docol * w1e_ref[...])
    t1 = s * (1.0 / (1.0 + jnp.exp(-s)))
    t2p = jnp.dot(t1.astype(jnp.bfloat16), w2_ref[...],
                  preferred_element_type=jnp.float32) + b2_ref[...]
    t2 = t2p * (1.0 / (1.0 + jnp.exp(-t2p)))
    t3t = lax.dot_general(w3_ref[...], t2.astype(jnp.bfloat16),
                          (((0,), (1,)), ((), ())),
                          preferred_element_type=jnp.float32)
    out_ref[...] = cdt_ref[...] * (jnp.tanh(t3t) * SCALE)


def _edge_mlp(g1, g2, d2d, do2d, cdt, w1d, w1e, W2, b2r, W3):
    grid = E_SEG // EBLK
    return pl.pallas_call(
        _edgemlp_body,
        grid=(grid,),
        in_specs=[
            pl.BlockSpec((EBLK, HIDDEN), lambda i: (i, 0)),
            pl.BlockSpec((EBLK, HIDDEN), lambda i: (i, 0)),
            pl.BlockSpec((1, DROWS, HIDDEN), lambda i: (i, 0, 0)),
            pl.BlockSpec((1, DROWS, HIDDEN), lambda i: (i, 0, 0)),
            pl.BlockSpec((3, EBLK), lambda i: (0, i)),
            pl.BlockSpec((1, HIDDEN), lambda i: (0, 0)),
            pl.BlockSpec((1, HIDDEN), lambda i: (0, 0)),
            pl.BlockSpec((HIDDEN, HIDDEN), lambda i: (0, 0)),
            pl.BlockSpec((1, HIDDEN), lambda i: (0, 0)),
            pl.BlockSpec((HIDDEN, 1), lambda i: (0, 0)),
            pl.BlockSpec((HIDDEN, HIDDEN), lambda i: (0, 0)),
        ],
        out_specs=pl.BlockSpec((3, EBLK), lambda i: (0, i)),
        out_shape=jax.ShapeDtypeStruct((3, E_SEG), jnp.float32),
    )(g1, g2, d2d, do2d, cdt, w1d, w1e, W2, b2r, W3,
      jnp.eye(HIDDEN, dtype=jnp.float32))


def _scatter_body(trans_hbm, row_hbm, zero_hbm, out_hbm, idx_v, t_v, acc_v):
    cid = lax.axis_index("c")
    sid = lax.axis_index("s")
    w = sid * NC + cid

    pltpu.sync_copy(zero_hbm, acc_v)

    def step(j, carry):
        c = w + NW * j

        @pl.when(c < NCHUNKS)
        def _():
            base = c * CHUNK
            pltpu.sync_copy(row_hbm.at[pl.ds(base, CHUNK)], idx_v)
            pltpu.sync_copy(trans_hbm.at[:, pl.ds(base, CHUNK)], t_v)
            lane = lax.iota(jnp.int32, 16)
            for k in range(CHUNK // 16):
                row16 = idx_v[pl.ds(k * 16, 16)]
                for comp in range(3):
                    vals = t_v[comp, pl.ds(k * 16, 16)]
                    plsc.addupdate_scatter(
                        acc_v, [row16 + (comp * NP_PAD)], vals)

        return carry

    lax.fori_loop(0, ITERS, step, 0)

    pltpu.sync_copy(acc_v, out_hbm.at[pl.ds(w * 3 * NP_PAD, 3 * NP_PAD)])


def _sc_scatter(trans, row, zeros_flat):
    mesh = plsc.VectorSubcoreMesh(core_axis_name="c", subcore_axis_name="s", num_cores=NC, num_subcores=NS)
    kern = pl.kernel(
        _scatter_body,
        out_type=jax.ShapeDtypeStruct((NW * 3 * NP_PAD,), jnp.float32),
        mesh=mesh,
        compiler_params=pltpu.CompilerParams(needs_layout_passes=False),
        scratch_types=[
            pltpu.VMEM((CHUNK,), jnp.int32),
            pltpu.VMEM((3, CHUNK), jnp.float32),
            pltpu.VMEM((3 * NP_PAD,), jnp.float32),
        ],
    )
    return kern(trans, row, zeros_flat)


def _reduce_body(p_ref, xt_ref, out_ref):
    out_ref[...] = xt_ref[...] + jnp.sum(p_ref[...], axis=0)


def _tc_reduce(partials, xt):
    blk = 2048
    grid = NP_PAD // blk
    return pl.pallas_call(
        _reduce_body,
        grid=(grid,),
        in_specs=[
            pl.BlockSpec((NW, 3, blk), lambda i: (0, 0, i)),
            pl.BlockSpec((3, blk), lambda i: (0, i)),
        ],
        out_specs=pl.BlockSpec((3, blk), lambda i: (0, i)),
        out_shape=jax.ShapeDtypeStruct((3, NP_PAD), jnp.float32),
    )(partials, xt)


@jax.jit
def kernel(h, x, edges, coord_diff, distances, distance_org, W1, b1, W2, b2, W3):
    row = edges[0].astype(jnp.int32)
    col = edges[1].astype(jnp.int32)
    W2 = W2.astype(jnp.bfloat16)
    W3 = W3.astype(jnp.bfloat16)

    w1a = W1[:HIDDEN]
    w1b = W1[HIDDEN:2 * HIDDEN]
    w1d = W1[2 * HIDDEN].reshape(1, HIDDEN)
    w1e = W1[2 * HIDDEN + 1].reshape(1, HIDDEN)
    b1r = b1.reshape(1, HIDDEN)
    b2r = b2.reshape(1, HIDDEN)

    a, b = _node_proj(h, w1a, w1b, b1r)

    d2d = distances.reshape(N_EDGES // EBLK, DROWS, HIDDEN)
    do2d = distance_org.reshape(N_EDGES // EBLK, DROWS, HIDDEN)
    cdt = coord_diff.T
    bps = E_SEG // EBLK
    trans_parts = []
    for sgi in range(SEG):
        g1, g2 = _sc_gather(a, b,
                            lax.dynamic_slice_in_dim(row, sgi * E_SEG, E_SEG),
                            lax.dynamic_slice_in_dim(col, sgi * E_SEG, E_SEG))
        trans_parts.append(_edge_mlp(
            g1, g2,
            d2d[sgi * bps:(sgi + 1) * bps],
            do2d[sgi * bps:(sgi + 1) * bps],
            cdt[:, sgi * E_SEG:(sgi + 1) * E_SEG],
            w1d, w1e, W2, b2r, W3))
    trans = jnp.concatenate(trans_parts, axis=1)

    zeros_flat = jnp.zeros((3 * NP_PAD,), jnp.float32)
    partials = _sc_scatter(trans, row, zeros_flat)

    xt = jnp.pad(x.T, ((0, 0), (0, NP_PAD - N_NODES)))
    out = _tc_reduce(partials.reshape(NW, 3, NP_PAD), xt)
    return out[:, :N_NODES].T
